# Initial kernel scaffold; baseline (speedup 1.0000x reference)
import jax
import jax.numpy as jnp
from jax.experimental import pallas as pl


def kernel(vecs, codebook):
    raise NotImplementedError("write your pallas kernel here")


# speedup vs baseline: 5.5844x; 5.5844x over previous
import functools

import jax
import jax.numpy as jnp
from jax import lax
from jax.experimental import pallas as pl
from jax.experimental.pallas import tpu as pltpu
from jax.experimental.pallas import tpu_sc as plsc

B_BLK = 256
NW = 32
CHUNK = 128


def _assign_body(v_ref, cbt_ref, idx_ref):
    P, D, K = cbt_ref.shape
    iota_k = lax.broadcasted_iota(jnp.int32, (B_BLK, K), 1)
    for p in range(P):
        cp = cbt_ref[p]
        half_cn = 0.5 * jnp.sum(cp * cp, axis=0, keepdims=True)
        vp = v_ref[:, p * D:(p + 1) * D]
        s = lax.dot_general(vp, cp, (((1,), (0,)), ((), ())),
                            precision=lax.Precision.HIGHEST,
                            preferred_element_type=jnp.float32) - half_cn
        m = jnp.max(s, axis=1, keepdims=True)
        am = jnp.min(jnp.where(s >= m, iota_k, K), axis=1)
        idx_ref[:, p] = am.astype(jnp.int32) + p * K


def _sc_gather(table, idx3, d):
    nw, c, chunk = idx3.shape
    b_per_w = c * chunk
    mesh = plsc.VectorSubcoreMesh(core_axis_name="c", subcore_axis_name="s")

    @functools.partial(
        pl.kernel,
        out_type=jax.ShapeDtypeStruct((nw * b_per_w, d), jnp.float32),
        mesh=mesh,
        scratch_types=[
            pltpu.VMEM((c, chunk), jnp.int32),
            pltpu.VMEM((b_per_w, d), jnp.float32),
            pltpu.SemaphoreType.DMA,
        ],
        compiler_params=pltpu.CompilerParams(use_tc_tiling_on_sc=False),
    )
    def gather_kernel(table_hbm, idx_hbm, out_hbm, idx_v, rows_v, sem):
        wid = lax.axis_index("s") * 2 + lax.axis_index("c")
        pltpu.sync_copy(idx_hbm.at[wid], idx_v)

        def step(s_, carry):
            copies = [
                pltpu.async_copy(
                    table_hbm.at[idx_v.at[s_ * 8 + i]],
                    rows_v.at[pl.ds((s_ * 8 + i) * chunk, chunk)],
                    sem,
                )
                for i in range(8)
            ]
            for cp in copies:
                cp.wait()
            return carry

        lax.fori_loop(0, c // 8, step, 0)
        pltpu.sync_copy(rows_v, out_hbm.at[pl.ds(wid * b_per_w, b_per_w)])

    return gather_kernel(table, idx3)


def kernel(vecs, codebook):
    B, E = vecs.shape
    P, K, D = codebook.shape
    cbt = jnp.transpose(codebook, (0, 2, 1))

    flat_idx = pl.pallas_call(
        _assign_body,
        grid=(B // B_BLK,),
        in_specs=[
            pl.BlockSpec((B_BLK, E), lambda j: (j, 0)),
            pl.BlockSpec((P, D, K), lambda j: (0, 0, 0)),
        ],
        out_specs=pl.BlockSpec((B_BLK, P), lambda j: (j, 0)),
        out_shape=jax.ShapeDtypeStruct((B, P), jnp.int32),
    )(vecs, cbt)

    table = codebook.reshape(P * K, D)
    idx3 = flat_idx.reshape(NW, (B * P) // (NW * CHUNK), CHUNK)
    rows = _sc_gather(table, idx3, D)
    return rows.reshape(B, P * D)
